# baseline (speedup 1.0000x reference)
import functools

import jax
import jax.numpy as jnp
from jax import lax
from jax.experimental import pallas as pl
from jax.experimental.pallas import tpu as pltpu
from jax.experimental.pallas import tpu_sc as plsc

E = 64
L = 16
NC = 2
NS = 16
NW = NC * NS
MARGIN = 1.0
ALPHA = 0.2
EPS = 1e-7
DU = 4


def _sc_body(pu, pi, php, ptp, phn, ptn, cu_a, ci_a, chp_a, ctp_a, chn_a,
             ctn_a, rp, rn, u_emb, e_emb, r_emb,
             s_out, sqp_out, sqn_out,
             idx_u, idx_i, idx_hp, idx_tp, idx_hn, idx_tn,
             col_u, col_i, col_hp, col_tp, col_hn, col_tn, idx_rp, idx_rn,
             r_tab,
             rows_u0, rows_i0, rows_hp0, rows_tp0, rows_hn0, rows_tn0,
             rows_u1, rows_i1, rows_hp1, rows_tp1, rows_hn1, rows_tn1,
             svec, pvec, nvec, sem0, sem1, isem):
    bpw = svec.shape[0]
    c_rows = rows_u0.shape[0]
    n_chunks = bpw // c_rows
    groups = c_rows // L

    wid = lax.axis_index("s") * NC + lax.axis_index("c")
    base = wid * bpw

    sl_w = pl.ds(base, bpw)
    stage = [
        pltpu.async_copy(pu.at[sl_w], idx_u, isem),
        pltpu.async_copy(pi.at[sl_w], idx_i, isem),
        pltpu.async_copy(php.at[sl_w], idx_hp, isem),
        pltpu.async_copy(ptp.at[sl_w], idx_tp, isem),
        pltpu.async_copy(phn.at[sl_w], idx_hn, isem),
        pltpu.async_copy(ptn.at[sl_w], idx_tn, isem),
        pltpu.async_copy(cu_a.at[sl_w], col_u, isem),
        pltpu.async_copy(ci_a.at[sl_w], col_i, isem),
        pltpu.async_copy(chp_a.at[sl_w], col_hp, isem),
        pltpu.async_copy(ctp_a.at[sl_w], col_tp, isem),
        pltpu.async_copy(chn_a.at[sl_w], col_hn, isem),
        pltpu.async_copy(ctn_a.at[sl_w], col_tn, isem),
        pltpu.async_copy(rp.at[sl_w], idx_rp, isem),
        pltpu.async_copy(rn.at[sl_w], idx_rn, isem),
        pltpu.async_copy(r_emb, r_tab, isem),
    ]
    for cp in stage:
        cp.wait()

    bufs = (
        (rows_u0, rows_i0, rows_hp0, rows_tp0, rows_hn0, rows_tn0, sem0),
        (rows_u1, rows_i1, rows_hp1, rows_tp1, rows_hn1, rows_tn1, sem1),
    )

    def issue(c):
        ru, ri, rhp, rtp, rhn, rtn, sem = bufs[c % 2]
        sl = pl.ds(c * c_rows, c_rows)
        return [
            pltpu.async_copy(u_emb.at[idx_u.at[sl]], ru, sem),
            pltpu.async_copy(e_emb.at[idx_i.at[sl]], ri, sem),
            pltpu.async_copy(e_emb.at[idx_hp.at[sl]], rhp, sem),
            pltpu.async_copy(e_emb.at[idx_tp.at[sl]], rtp, sem),
            pltpu.async_copy(e_emb.at[idx_hn.at[sl]], rhn, sem),
            pltpu.async_copy(e_emb.at[idx_tn.at[sl]], rtn, sem),
        ]

    iota = lax.iota(jnp.int32, L)
    pend = {0: issue(0)}
    for c in range(n_chunks):
        if c + 1 < n_chunks:
            pend[c + 1] = issue(c + 1)
        with jax.named_scope(f"dma_wait_{c}"):
            for cp in pend.pop(c):
                cp.wait()
        ru, ri, rhp, rtp, rhn, rtn, _ = bufs[c % 2]
        cbase = c * c_rows

        def group_body(g, _, ru=ru, ri=ri, rhp=rhp, rtp=rtp, rhn=rhn,
                       rtn=rtn, cbase=cbase):
            goff = pl.multiple_of(g * L, L)
            row = goff + iota
            sl16 = pl.ds(cbase + goff, L)
            rp_v = idx_rp[sl16]
            rn_v = idx_rn[sl16]
            cu = col_u[sl16]
            ci = col_i[sl16]
            chp = col_hp[sl16]
            ctp = col_tp[sl16]
            chn = col_hn[sl16]
            ctn = col_tn[sl16]

            def d_body(dd, accs):
                acc_s, acc_p, acc_n = accs
                for k in range(DU):
                    d = dd * DU + k
                    col = jnp.full((L,), d, jnp.int32)
                    ue = plsc.load_gather(ru, [row, cu + d])
                    ie = plsc.load_gather(ri, [row, ci + d])
                    acc_s = acc_s + ue * ie
                    hpe = plsc.load_gather(rhp, [row, chp + d])
                    tpe = plsc.load_gather(rtp, [row, ctp + d])
                    rpe = plsc.load_gather(r_tab, [rp_v, col])
                    dp = hpe + rpe - tpe
                    acc_p = acc_p + dp * dp
                    hne = plsc.load_gather(rhn, [row, chn + d])
                    tne = plsc.load_gather(rtn, [row, ctn + d])
                    rne = plsc.load_gather(r_tab, [rn_v, col])
                    dn = hne + rne - tne
                    acc_n = acc_n + dn * dn
                return acc_s, acc_p, acc_n

            zero = jnp.zeros((L,), jnp.float32)
            acc_s, acc_p, acc_n = lax.fori_loop(0, E // DU, d_body,
                                                (zero, zero, zero))
            svec[sl16] = acc_s
            pvec[sl16] = acc_p
            nvec[sl16] = acc_n
            return 0

        with jax.named_scope(f"compute_{c}"):
            lax.fori_loop(0, groups, group_body, 0)

    pltpu.sync_copy(svec, s_out.at[sl_w])
    pltpu.sync_copy(pvec, sqp_out.at[sl_w])
    pltpu.sync_copy(nvec, sqn_out.at[sl_w])


def _make_sc_call(batch):
    bpw = batch // NW
    c_rows = min(bpw, 64)
    mesh = plsc.VectorSubcoreMesh(core_axis_name="c", subcore_axis_name="s")
    f32 = jnp.float32
    return pl.kernel(
        _sc_body,
        out_type=[jax.ShapeDtypeStruct((batch,), f32)] * 3,
        mesh=mesh,
        compiler_params=pltpu.CompilerParams(
            needs_layout_passes=False, use_tc_tiling_on_sc=True),
        scratch_types=(
            [pltpu.VMEM((bpw,), jnp.int32)] * 14
            + [pltpu.VMEM((64, E), f32)]
            + [pltpu.VMEM((c_rows, 2 * E), f32)] * 12
            + [pltpu.VMEM((bpw,), f32)] * 3
            + [pltpu.SemaphoreType.DMA] * 3
        ),
    )


def _finish_body(y_ref, s_ref, p_ref, n_ref, o_ref):
    s = s_ref[...]
    yp = jnp.clip(1.0 / (1.0 + jnp.exp(-s)), EPS, 1.0 - EPS)
    yv = y_ref[...]
    bce = -(yv * jnp.log(yp) + (1.0 - yv) * jnp.log(1.0 - yp))
    ypos = jnp.sqrt(p_ref[...])
    yneg = jnp.sqrt(n_ref[...])
    hinge = jnp.maximum(ypos - yneg + MARGIN, 0.0)
    n = s.shape[0] * s.shape[1]
    o_ref[0, 0] = jnp.sum(bce) / n + ALPHA * jnp.sum(hinge)


def kernel(u, i, y, h_pos, r_pos, t_pos, h_neg, r_neg, t_neg, u_emb, e_emb, r_emb):
    batch = u.shape[0]
    n_user, e_dim = u_emb.shape
    n_ent = e_emb.shape[0]
    u2 = u_emb.reshape(n_user // 2, 2 * e_dim)
    e2 = e_emb.reshape(n_ent // 2, 2 * e_dim)

    def prep(idx):
        idx = idx.astype(jnp.int32)
        return idx >> 1, (idx & 1) * e_dim

    pu, cu = prep(u)
    pi, ci = prep(i)
    php, chp = prep(h_pos)
    ptp, ctp = prep(t_pos)
    phn, chn = prep(h_neg)
    ptn, ctn = prep(t_neg)

    sc_call = _make_sc_call(batch)
    s, sqp, sqn = sc_call(pu, pi, php, ptp, phn, ptn, cu, ci, chp, ctp, chn,
                          ctn, r_pos.astype(jnp.int32), r_neg.astype(jnp.int32),
                          u2, e2, r_emb)
    rows = batch // 128
    shape2d = (rows, 128)
    out = pl.pallas_call(
        _finish_body,
        out_shape=jax.ShapeDtypeStruct((1, 1), jnp.float32),
        out_specs=pl.BlockSpec(memory_space=pltpu.SMEM),
    )(y.reshape(shape2d), s.reshape(shape2d), sqp.reshape(shape2d),
      sqn.reshape(shape2d))
    return out[0, 0]
